# baseline (speedup 1.0000x reference)
import functools

import numpy as np
import jax
import jax.numpy as jnp
from jax import lax
from jax.experimental import pallas as pl
from jax.experimental.pallas import tpu as pltpu
from jax.experimental.pallas import tpu_sc as plsc

B, T, L_OUT = 16, 512, 2046
LP = 2048
D_TEXT, D_EMO, D_SPK = 256, 32, 32
C_IN = D_TEXT + D_EMO + D_SPK
M, F, FILT = 128, 256, 11
NL = 3
G4 = 512
DTAB = 256
NROWS = B * LP
NEG_LOG1E4 = float(-np.log(10000.0))

f32 = jnp.float32


def _dot(a, b):
    return lax.dot_general(a, b, (((1,), (0,)), ((), ())),
                           preferred_element_type=f32)


def _dot_t(a, b):
    return lax.dot_general(a, b, (((1,), (1,)), ((), ())),
                           preferred_element_type=f32)


def _relu(x):
    return jnp.maximum(x, 0.0)


def _k1_body(*refs):
    it = iter(refs)
    text_ref, emo_ref, spk_ref = next(it), next(it), next(it)
    dur_ref, pit_ref, ene_ref = next(it), next(it), next(it)
    pe_w, pe_b, ee_w, ee_b = next(it), next(it), next(it), next(it)
    pred_w = [[next(it) for _ in range(19)] for _ in range(2)]
    wp1, bp1, wp2, bp2, wih_d, bd = (next(it) for _ in range(6))
    xwp_ref, xwe_ref, xwd_ref = next(it), next(it), next(it)
    tab_ref, src_ref, tot_ref, len_ref = next(it), next(it), next(it), next(it)
    lre_ref, lrs_ref, sidx_ref = next(it), next(it), next(it)
    pad_ref = next(it)

    b_id = pl.program_id(0)
    text = text_ref[0]
    emoT = emo_ref[0]
    spkT = spk_ref[0]

    i0 = lax.broadcasted_iota(jnp.int32, (T, T), 0)
    i1 = lax.broadcasted_iota(jnp.int32, (T, T), 1)
    ident = (i0 == i1).astype(f32)
    tri = (i1 <= i0).astype(f32)
    emo = _dot_t(ident, emoT)
    spk = _dot_t(ident, spkT)

    def conv9(col, w_ref, b_ref):
        pad_ref[0:8, 0:1] = jnp.zeros((8, 1), f32)
        pad_ref[8:8 + T, 0:1] = col
        pad_ref[8 + T:16 + T, 0:1] = jnp.zeros((8, 1), f32)
        w = w_ref[...]
        acc = jnp.broadcast_to(b_ref[...], (T, D_TEXT))
        for k in range(9):
            acc = acc + pad_ref[4 + k:4 + k + T, 0:1] * w[k:k + 1, :]
        return acc

    pe = conv9(_dot_t(ident, pit_ref[pl.ds(b_id, 1), :]), pe_w, pe_b)
    ee = conv9(_dot_t(ident, ene_ref[pl.ds(b_id, 1), :]), ee_w, ee_b)
    text_aug = text + pe + ee

    def fsmn(w):
        wi, bi = w[0], w[1]
        h = _relu(_dot(text, wi[0:256, :]) + _dot(spk, wi[256:288, :])
                  + _dot(emo, wi[288:320, :]) + bi[...])
        for l in range(NL):
            mem, w1, b1, w2, b2 = w[2 + 5 * l:7 + 5 * l]
            pad_ref[0:8, :] = jnp.zeros((8, M), f32)
            pad_ref[8:8 + T, :] = h
            pad_ref[8 + T:16 + T, :] = jnp.zeros((8, M), f32)
            memv = mem[...]
            conv = jnp.zeros((T, M), f32)
            for k in range(FILT):
                conv = conv + pad_ref[3 + k:3 + k + T, :] * memv[k:k + 1, :]
            h2 = h + conv
            h = h2 + _dot(_relu(_dot(h2, w1[...]) + b1[...]), w2[...]) + b2[...]
        return _dot(h, w[17][...])

    xwp_ref[...] = fsmn(pred_w[0]).reshape(1, T, G4)
    xwe_ref[...] = fsmn(pred_w[1]).reshape(1, T, G4)

    dur_row = dur_ref[pl.ds(b_id, 1), :].astype(f32)
    dur_f = _dot_t(ident, dur_row)
    pad_ref[0:8, 0:1] = jnp.zeros((8, 1), f32)
    pad_ref[8:8 + T, 0:1] = dur_f
    dur_prev = pad_ref[7:7 + T, 0:1]
    dur_in = jnp.log(dur_prev + 1.0)
    h = _relu(dur_in * wp1[0:1, :] + _dot(text_aug, wp1[1:257, :])
              + _dot(spk, wp1[257:289, :]) + _dot(emo, wp1[289:321, :])
              + bp1[...])
    h = _relu(_dot(h, wp2[...]) + bp2[...])
    xwd_ref[...] = _dot(h, wih_d[...]).reshape(1, T, G4)

    cums = _dot(tri, dur_f)
    start_col = cums - dur_f

    tab_ref[...] = text_aug

    pos = lax.broadcasted_iota(jnp.int32, (1, LP), 1).astype(f32)
    cmp_c = (cums <= pos).astype(f32)
    cnt = jnp.sum(cmp_c, axis=0, keepdims=True)
    src = jnp.minimum(cnt, float(T - 1)).astype(jnp.int32)
    src_ref[...] = (src + b_id * T).reshape(1, 1, LP)

    start_at_p = _dot(dur_row, cmp_c)
    local = pos - start_at_p + 1.0
    sidx_ref[...] = jnp.clip(local, 0.0, 7.0).astype(jnp.int32).reshape(1, 1, LP)

    oh = (start_col <= pos).astype(f32) - cmp_c
    ohs = oh[:, 0:L_OUT]
    lre_ref[...] = _dot(emoT, ohs).reshape(1, D_EMO, L_OUT)
    lrs_ref[...] = _dot(spkT, ohs).reshape(1, D_SPK, L_OUT)

    total = jnp.sum(dur_f)
    tot_ref[...] = jnp.broadcast_to(total, (1, 1, 1))
    len_ref[...] = jnp.broadcast_to(
        jnp.minimum(jnp.ceil(total / 3.0) * 3.0, float(L_OUT)), (1, 1, 1))


def _run_k1(text, emo, spk, dur_tb, pit_tb, ene_tb, weights):
    in_specs = [
        pl.BlockSpec((1, T, D_TEXT), lambda b: (b, 0, 0)),
        pl.BlockSpec((1, D_EMO, T), lambda b: (b, 0, 0)),
        pl.BlockSpec((1, D_SPK, T), lambda b: (b, 0, 0)),
        pl.BlockSpec((B, T), lambda b: (0, 0)),
        pl.BlockSpec((B, T), lambda b: (0, 0)),
        pl.BlockSpec((B, T), lambda b: (0, 0)),
    ] + [pl.BlockSpec(w.shape, functools.partial(lambda n, b: (0,) * n, w.ndim))
         for w in weights]
    out_shape = [
        jax.ShapeDtypeStruct((B, T, G4), f32),
        jax.ShapeDtypeStruct((B, T, G4), f32),
        jax.ShapeDtypeStruct((B, T, G4), f32),
        jax.ShapeDtypeStruct((B * T, DTAB), f32),
        jax.ShapeDtypeStruct((B, 1, LP), jnp.int32),
        jax.ShapeDtypeStruct((B, 1, 1), f32),
        jax.ShapeDtypeStruct((B, 1, 1), f32),
        jax.ShapeDtypeStruct((B, D_EMO, L_OUT), f32),
        jax.ShapeDtypeStruct((B, D_SPK, L_OUT), f32),
        jax.ShapeDtypeStruct((B, 1, LP), jnp.int32),
    ]
    out_specs = [
        pl.BlockSpec((1, T, G4), lambda b: (b, 0, 0)),
        pl.BlockSpec((1, T, G4), lambda b: (b, 0, 0)),
        pl.BlockSpec((1, T, G4), lambda b: (b, 0, 0)),
        pl.BlockSpec((T, DTAB), lambda b: (b, 0)),
        pl.BlockSpec((1, 1, LP), lambda b: (b, 0, 0)),
        pl.BlockSpec((1, 1, 1), lambda b: (b, 0, 0)),
        pl.BlockSpec((1, 1, 1), lambda b: (b, 0, 0)),
        pl.BlockSpec((1, D_EMO, L_OUT), lambda b: (b, 0, 0)),
        pl.BlockSpec((1, D_SPK, L_OUT), lambda b: (b, 0, 0)),
        pl.BlockSpec((1, 1, LP), lambda b: (b, 0, 0)),
    ]
    return pl.pallas_call(
        _k1_body,
        grid=(B,),
        in_specs=in_specs,
        out_specs=out_specs,
        out_shape=out_shape,
        scratch_shapes=[pltpu.VMEM((T + 16, M), f32)],
        compiler_params=pltpu.CompilerParams(
            dimension_semantics=("arbitrary",)),
    )(text, emo, spk, dur_tb, pit_tb, ene_tb, *weights)


_SC_CHUNK = 128
_SC_NW = 32
_SC_PER_W = NROWS // _SC_NW


@functools.lru_cache(maxsize=None)
def _sc_gather_fn():
    mesh = plsc.VectorSubcoreMesh(core_axis_name="c", subcore_axis_name="s")
    nch = _SC_PER_W // _SC_CHUNK

    @functools.partial(
        pl.kernel,
        out_type=jax.ShapeDtypeStruct((NROWS, DTAB), f32),
        mesh=mesh,
        scratch_types=[
            pltpu.VMEM((_SC_PER_W,), jnp.int32),
            pltpu.VMEM((_SC_CHUNK, DTAB), f32),
            pltpu.VMEM((_SC_CHUNK, DTAB), f32),
            pltpu.SemaphoreType.DMA,
            pltpu.SemaphoreType.DMA,
            pltpu.SemaphoreType.DMA,
            pltpu.SemaphoreType.DMA,
        ],
    )
    def _sc_gather(tab_hbm, idx_hbm, out_hbm, idx_v, rows0, rows1,
                   gsem0, gsem1, ssem0, ssem1):
        wid = lax.axis_index("s") * 2 + lax.axis_index("c")
        base0 = wid * _SC_PER_W
        rows = [rows0, rows1]
        gsem = [gsem0, gsem1]
        ssem = [ssem0, ssem1]

        pltpu.sync_copy(idx_hbm.at[pl.ds(base0, _SC_PER_W)], idx_v)

        gd = [None] * nch
        sd = [None] * nch
        for i in range(nch):
            b = i & 1
            if i >= 2:
                sd[i - 2].wait()
            gd[i] = pltpu.async_copy(
                tab_hbm.at[idx_v.at[pl.ds(i * _SC_CHUNK, _SC_CHUNK)]],
                rows[b], gsem[b])
            if i >= 1:
                gd[i - 1].wait()
                sd[i - 1] = pltpu.async_copy(
                    rows[1 - b],
                    out_hbm.at[pl.ds(base0 + (i - 1) * _SC_CHUNK, _SC_CHUNK)],
                    ssem[1 - b])
        gd[nch - 1].wait()
        last = nch - 1
        sd[last] = pltpu.async_copy(
            rows[last & 1],
            out_hbm.at[pl.ds(base0 + last * _SC_CHUNK, _SC_CHUNK)],
            ssem[last & 1])
        sd[nch - 2].wait()
        sd[last].wait()

    return _sc_gather


NT = 128


def _k3_body(xwp_ref, xwe_ref, xwd_ref, whp_ref, whe_ref, whd_ref,
             blp_ref, ble_ref, bld_ref,
             wo_ref, bo_ref, pp_ref, pe_ref, pd_ref, h_s, c_s, ys_ref):
    @pl.when(pl.program_id(0) == 0)
    def _init():
        h_s[...] = jnp.zeros_like(h_s)
        c_s[...] = jnp.zeros_like(c_s)

    whp, whe, whd = whp_ref[...], whe_ref[...], whd_ref[...]
    blp, ble, bld = blp_ref[...], ble_ref[...], bld_ref[...]

    def step(t, carry):
        h, c = carry
        g = jnp.concatenate([
            (xwp_ref[:, t, :] + _dot(h[0:16, :], whp)) + blp,
            (xwe_ref[:, t, :] + _dot(h[16:32, :], whe)) + ble,
            (xwd_ref[:, t, :] + _dot(h[32:48, :], whd)) + bld,
        ], axis=0)
        ig = jax.nn.sigmoid(g[:, 0:128])
        fg = jax.nn.sigmoid(g[:, 128:256])
        gg = jnp.tanh(g[:, 256:384])
        og = jax.nn.sigmoid(g[:, 384:512])
        c2 = fg * c + ig * gg
        h2 = og * jnp.tanh(c2)
        ys_ref[t] = h2
        return (h2, c2)

    h, c = lax.fori_loop(0, NT, step, (h_s[...], c_s[...]), unroll=16)
    h_s[...] = h
    c_s[...] = c

    ys = ys_ref[...]
    pred = jnp.sum(ys * wo_ref[...][None, :, :], axis=2) + bo_ref[...]
    pp_ref[...] = pred[:, 0:16]
    pe_ref[...] = pred[:, 16:32]
    pd_ref[...] = pred[:, 32:48]


def _run_k3(xwp, xwe, xwd, whp, whe, whd, blp, ble, bld, wo_stack, bo_stack):
    xw_spec = pl.BlockSpec((B, NT, G4), lambda i: (0, i, 0))
    w_spec = pl.BlockSpec((M, G4), lambda i: (0, 0))
    b_spec = pl.BlockSpec((1, G4), lambda i: (0, 0))
    p_spec = pl.BlockSpec((NT, B), lambda i: (i, 0))
    return pl.pallas_call(
        _k3_body,
        grid=(T // NT,),
        in_specs=[xw_spec, xw_spec, xw_spec, w_spec, w_spec, w_spec,
                  b_spec, b_spec, b_spec,
                  pl.BlockSpec((3 * B, M), lambda i: (0, 0)),
                  pl.BlockSpec((1, 3 * B), lambda i: (0, 0))],
        out_specs=[p_spec, p_spec, p_spec],
        out_shape=[jax.ShapeDtypeStruct((T, B), f32)] * 3,
        scratch_shapes=[pltpu.VMEM((3 * B, M), f32),
                        pltpu.VMEM((3 * B, M), f32),
                        pltpu.VMEM((NT, 3 * B, M), f32)],
        compiler_params=pltpu.CompilerParams(
            dimension_semantics=("arbitrary",)),
    )(xwp, xwe, xwd, whp, whe, whd, blp, ble, bld, wo_stack, bo_stack)


NCH4 = 8
CH4 = LP // NCH4


def _k4_body(g_ref, sidx_ref, tot_ref, lrt_ref):
    g = g_ref[...]
    c = pl.program_id(0)
    pos3 = (lax.broadcasted_iota(jnp.int32, (CH4, B, 1), 0)
            + c * CH4).astype(f32)
    mask3 = (pos3 < tot_ref[...]).astype(f32)
    mask = mask3.reshape(CH4 * B, 1)
    k = lax.broadcasted_iota(jnp.int32, (1, D_TEXT // 2), 1).astype(f32)
    freqs = jnp.exp(k * NEG_LOG1E4 / float(D_TEXT // 2))
    l8 = lax.broadcasted_iota(jnp.int32, (8, 1), 0).astype(f32)
    ang8 = l8 * freqs
    stab = jnp.concatenate([jnp.sin(ang8), jnp.cos(ang8)], axis=1)
    onehot = (sidx_ref[...] ==
              lax.broadcasted_iota(jnp.int32, (1, 1, 8), 2)).astype(f32)
    sino = _dot(onehot.reshape(CH4 * B, 8), stab)
    lrt_ref[...] = ((g + sino) * mask).reshape(CH4, B, D_TEXT)


def _run_k4(g, sidx_pm, tot):
    return pl.pallas_call(
        _k4_body,
        grid=(NCH4,),
        in_specs=[pl.BlockSpec((CH4 * B, DTAB), lambda c: (c, 0)),
                  pl.BlockSpec((CH4, B, 1), lambda c: (c, 0, 0)),
                  pl.BlockSpec((1, B, 1), lambda c: (0, 0, 0))],
        out_specs=[pl.BlockSpec((CH4, B, D_TEXT), lambda c: (c, 0, 0))],
        out_shape=[jax.ShapeDtypeStruct((L_OUT, B, D_TEXT), f32)],
        compiler_params=pltpu.CompilerParams(
            dimension_semantics=("arbitrary",)),
    )(g, sidx_pm, tot)


def _pred_flat(pp):
    out = [pp['inp']['w'], pp['inp']['b'].reshape(1, M)]
    for lp in pp['layers']:
        out += [lp['mem'].reshape(FILT, M),
                lp['ffn1']['w'], lp['ffn1']['b'].reshape(1, F),
                lp['ffn2']['w'], lp['ffn2']['b'].reshape(1, M)]
    out += [pp['lstm']['W_ih'], pp['lstm']['b'].reshape(1, G4)]
    return out


def kernel(inputs_text_embedding, inputs_emo_embedding, inputs_spk_embedding,
           duration_targets, pitch_targets, energy_targets, params):
    text = inputs_text_embedding
    emo = jnp.swapaxes(inputs_emo_embedding, 1, 2)
    spk = jnp.swapaxes(inputs_spk_embedding, 1, 2)

    dur_tb = duration_targets.astype(jnp.int32)
    pit_tb = pitch_targets
    ene_tb = energy_targets

    weights = ([params['pitch_emb']['w'].reshape(9, D_TEXT),
                params['pitch_emb']['b'].reshape(1, D_TEXT),
                params['energy_emb']['w'].reshape(9, D_TEXT),
                params['energy_emb']['b'].reshape(1, D_TEXT)]
               + _pred_flat(params['pitch_pred'])
               + _pred_flat(params['energy_pred'])
               + [params['dur']['pre1']['w'],
                  params['dur']['pre1']['b'].reshape(1, M),
                  params['dur']['pre2']['w'],
                  params['dur']['pre2']['b'].reshape(1, M),
                  params['dur']['lstm']['W_ih'],
                  params['dur']['lstm']['b'].reshape(1, G4)])

    (xwp, xwe, xwd, table, srcflat, tot, lenr,
     lre_dp, lrs_dp, sidx) = _run_k1(
        text, emo, spk, dur_tb, pit_tb, ene_tb, weights)

    idx_pm = srcflat.reshape(B, LP).T.reshape(NROWS)
    g = _sc_gather_fn()(table, idx_pm)

    def _wo_row(pp):
        return jnp.broadcast_to(pp['out']['w'][:, 0][None, :], (B, M))

    wo_stack = jnp.concatenate([_wo_row(params['pitch_pred']),
                                _wo_row(params['energy_pred']),
                                _wo_row(params['dur'])], axis=0)
    bo_stack = jnp.concatenate(
        [jnp.broadcast_to(params['pitch_pred']['out']['b'].reshape(1, 1), (1, B)),
         jnp.broadcast_to(params['energy_pred']['out']['b'].reshape(1, 1), (1, B)),
         jnp.broadcast_to(params['dur']['out']['b'].reshape(1, 1), (1, B))],
        axis=1)

    ppt, pet, pdt = _run_k3(xwp, xwe, xwd,
                            params['pitch_pred']['lstm']['W_hh'],
                            params['energy_pred']['lstm']['W_hh'],
                            params['dur']['lstm']['W_hh'],
                            params['pitch_pred']['lstm']['b'].reshape(1, G4),
                            params['energy_pred']['lstm']['b'].reshape(1, G4),
                            params['dur']['lstm']['b'].reshape(1, G4),
                            wo_stack, bo_stack)

    sidx_pm = sidx.reshape(B, LP).T.reshape(LP, B, 1)
    lrt = _run_k4(g, sidx_pm, tot.reshape(1, B, 1))[0]

    return (jnp.swapaxes(lrt, 0, 1), jnp.swapaxes(lre_dp, 1, 2),
            jnp.swapaxes(lrs_dp, 1, 2), lenr.reshape(B), pdt.T, ppt.T, pet.T)


# speedup vs baseline: 1.0185x; 1.0185x over previous
import functools

import numpy as np
import jax
import jax.numpy as jnp
from jax import lax
from jax.experimental import pallas as pl
from jax.experimental.pallas import tpu as pltpu
from jax.experimental.pallas import tpu_sc as plsc

B, T, L_OUT = 16, 512, 2046
LP = 2048
D_TEXT, D_EMO, D_SPK = 256, 32, 32
C_IN = D_TEXT + D_EMO + D_SPK
M, F, FILT = 128, 256, 11
NL = 3
G4 = 512
DTAB = 256
NROWS = B * LP
NEG_LOG1E4 = float(-np.log(10000.0))

f32 = jnp.float32


def _dot(a, b):
    return lax.dot_general(a, b, (((1,), (0,)), ((), ())),
                           preferred_element_type=f32)


def _dot_t(a, b):
    return lax.dot_general(a, b, (((1,), (1,)), ((), ())),
                           preferred_element_type=f32)


def _relu(x):
    return jnp.maximum(x, 0.0)


def _k1_body(*refs):
    it = iter(refs)
    text_ref, emo_ref, spk_ref = next(it), next(it), next(it)
    dur_ref, pit_ref, ene_ref = next(it), next(it), next(it)
    pe_w, pe_b, ee_w, ee_b = next(it), next(it), next(it), next(it)
    pred_w = [[next(it) for _ in range(19)] for _ in range(2)]
    wp1, bp1, wp2, bp2, wih_d, bd = (next(it) for _ in range(6))
    xwp_ref, xwe_ref, xwd_ref = next(it), next(it), next(it)
    tab_ref, src_ref, tot_ref, len_ref = next(it), next(it), next(it), next(it)
    lre_ref, lrs_ref, sidx_ref = next(it), next(it), next(it)
    pad_ref = next(it)

    b_id = pl.program_id(0)
    text = text_ref[0]
    emoT = emo_ref[0]
    spkT = spk_ref[0]

    i0 = lax.broadcasted_iota(jnp.int32, (T, T), 0)
    i1 = lax.broadcasted_iota(jnp.int32, (T, T), 1)
    ident = (i0 == i1).astype(f32)
    tri = (i1 <= i0).astype(f32)
    emo = _dot_t(ident, emoT)
    spk = _dot_t(ident, spkT)

    def conv9(col, w_ref, b_ref):
        pad_ref[0:8, 0:1] = jnp.zeros((8, 1), f32)
        pad_ref[8:8 + T, 0:1] = col
        pad_ref[8 + T:16 + T, 0:1] = jnp.zeros((8, 1), f32)
        w = w_ref[...]
        acc = jnp.broadcast_to(b_ref[...], (T, D_TEXT))
        for k in range(9):
            acc = acc + pad_ref[4 + k:4 + k + T, 0:1] * w[k:k + 1, :]
        return acc

    pe = conv9(_dot_t(ident, pit_ref[pl.ds(b_id, 1), :]), pe_w, pe_b)
    ee = conv9(_dot_t(ident, ene_ref[pl.ds(b_id, 1), :]), ee_w, ee_b)
    text_aug = text + pe + ee

    def fsmn(w):
        wi, bi = w[0], w[1]
        h = _relu(_dot(text, wi[0:256, :]) + _dot(spk, wi[256:288, :])
                  + _dot(emo, wi[288:320, :]) + bi[...])
        for l in range(NL):
            mem, w1, b1, w2, b2 = w[2 + 5 * l:7 + 5 * l]
            pad_ref[0:8, :] = jnp.zeros((8, M), f32)
            pad_ref[8:8 + T, :] = h
            pad_ref[8 + T:16 + T, :] = jnp.zeros((8, M), f32)
            memv = mem[...]
            conv = jnp.zeros((T, M), f32)
            for k in range(FILT):
                conv = conv + pad_ref[3 + k:3 + k + T, :] * memv[k:k + 1, :]
            h2 = h + conv
            h = h2 + _dot(_relu(_dot(h2, w1[...]) + b1[...]), w2[...]) + b2[...]
        return _dot(h, w[17][...]) + w[18][...]

    xwp_ref[...] = fsmn(pred_w[0]).reshape(1, T, G4)
    xwe_ref[...] = fsmn(pred_w[1]).reshape(1, T, G4)

    dur_row = dur_ref[pl.ds(b_id, 1), :].astype(f32)
    dur_f = _dot_t(ident, dur_row)
    pad_ref[0:8, 0:1] = jnp.zeros((8, 1), f32)
    pad_ref[8:8 + T, 0:1] = dur_f
    dur_prev = pad_ref[7:7 + T, 0:1]
    dur_in = jnp.log(dur_prev + 1.0)
    h = _relu(dur_in * wp1[0:1, :] + _dot(text_aug, wp1[1:257, :])
              + _dot(spk, wp1[257:289, :]) + _dot(emo, wp1[289:321, :])
              + bp1[...])
    h = _relu(_dot(h, wp2[...]) + bp2[...])
    xwd_ref[...] = (_dot(h, wih_d[...]) + bd[...]).reshape(1, T, G4)

    cums = _dot(tri, dur_f)
    start_col = cums - dur_f

    tab_ref[...] = text_aug

    pos = lax.broadcasted_iota(jnp.int32, (1, LP), 1).astype(f32)
    cmp_c = (cums <= pos).astype(f32)
    cnt = jnp.sum(cmp_c, axis=0, keepdims=True)
    src = jnp.minimum(cnt, float(T - 1)).astype(jnp.int32)
    src_ref[...] = (src + b_id * T).reshape(1, 1, LP)

    start_at_p = _dot(dur_row, cmp_c)
    local = pos - start_at_p + 1.0
    sidx_ref[...] = jnp.clip(local, 0.0, 7.0).astype(jnp.int32).reshape(1, 1, LP)

    oh = (start_col <= pos).astype(f32) - cmp_c
    ohs = oh[:, 0:L_OUT]
    lre_ref[...] = _dot(emoT, ohs).reshape(1, D_EMO, L_OUT)
    lrs_ref[...] = _dot(spkT, ohs).reshape(1, D_SPK, L_OUT)

    total = jnp.sum(dur_f)
    tot_ref[...] = jnp.broadcast_to(total, (1, 1, 1))
    len_ref[...] = jnp.broadcast_to(
        jnp.minimum(jnp.ceil(total / 3.0) * 3.0, float(L_OUT)), (1, 1, 1))


def _run_k1(text, emo, spk, dur_tb, pit_tb, ene_tb, weights):
    in_specs = [
        pl.BlockSpec((1, T, D_TEXT), lambda b: (b, 0, 0)),
        pl.BlockSpec((1, D_EMO, T), lambda b: (b, 0, 0)),
        pl.BlockSpec((1, D_SPK, T), lambda b: (b, 0, 0)),
        pl.BlockSpec((B, T), lambda b: (0, 0)),
        pl.BlockSpec((B, T), lambda b: (0, 0)),
        pl.BlockSpec((B, T), lambda b: (0, 0)),
    ] + [pl.BlockSpec(w.shape, functools.partial(lambda n, b: (0,) * n, w.ndim))
         for w in weights]
    out_shape = [
        jax.ShapeDtypeStruct((B, T, G4), f32),
        jax.ShapeDtypeStruct((B, T, G4), f32),
        jax.ShapeDtypeStruct((B, T, G4), f32),
        jax.ShapeDtypeStruct((B * T, DTAB), f32),
        jax.ShapeDtypeStruct((B, 1, LP), jnp.int32),
        jax.ShapeDtypeStruct((B, 1, 1), f32),
        jax.ShapeDtypeStruct((B, 1, 1), f32),
        jax.ShapeDtypeStruct((B, D_EMO, L_OUT), f32),
        jax.ShapeDtypeStruct((B, D_SPK, L_OUT), f32),
        jax.ShapeDtypeStruct((B, 1, LP), jnp.int32),
    ]
    out_specs = [
        pl.BlockSpec((1, T, G4), lambda b: (b, 0, 0)),
        pl.BlockSpec((1, T, G4), lambda b: (b, 0, 0)),
        pl.BlockSpec((1, T, G4), lambda b: (b, 0, 0)),
        pl.BlockSpec((T, DTAB), lambda b: (b, 0)),
        pl.BlockSpec((1, 1, LP), lambda b: (b, 0, 0)),
        pl.BlockSpec((1, 1, 1), lambda b: (b, 0, 0)),
        pl.BlockSpec((1, 1, 1), lambda b: (b, 0, 0)),
        pl.BlockSpec((1, D_EMO, L_OUT), lambda b: (b, 0, 0)),
        pl.BlockSpec((1, D_SPK, L_OUT), lambda b: (b, 0, 0)),
        pl.BlockSpec((1, 1, LP), lambda b: (b, 0, 0)),
    ]
    return pl.pallas_call(
        _k1_body,
        grid=(B,),
        in_specs=in_specs,
        out_specs=out_specs,
        out_shape=out_shape,
        scratch_shapes=[pltpu.VMEM((T + 16, M), f32)],
        compiler_params=pltpu.CompilerParams(
            dimension_semantics=("arbitrary",)),
    )(text, emo, spk, dur_tb, pit_tb, ene_tb, *weights)


_SC_CHUNK = 128
_SC_NW = 32
_SC_PER_W = NROWS // _SC_NW


@functools.lru_cache(maxsize=None)
def _sc_gather_fn():
    mesh = plsc.VectorSubcoreMesh(core_axis_name="c", subcore_axis_name="s")
    nch = _SC_PER_W // _SC_CHUNK

    @functools.partial(
        pl.kernel,
        out_type=jax.ShapeDtypeStruct((NROWS, DTAB), f32),
        mesh=mesh,
        scratch_types=[
            pltpu.VMEM((_SC_PER_W,), jnp.int32),
            pltpu.VMEM((_SC_CHUNK, DTAB), f32),
            pltpu.VMEM((_SC_CHUNK, DTAB), f32),
            pltpu.SemaphoreType.DMA,
            pltpu.SemaphoreType.DMA,
            pltpu.SemaphoreType.DMA,
            pltpu.SemaphoreType.DMA,
        ],
    )
    def _sc_gather(tab_hbm, idx_hbm, out_hbm, idx_v, rows0, rows1,
                   gsem0, gsem1, ssem0, ssem1):
        wid = lax.axis_index("s") * 2 + lax.axis_index("c")
        base0 = wid * _SC_PER_W
        rows = [rows0, rows1]
        gsem = [gsem0, gsem1]
        ssem = [ssem0, ssem1]

        pltpu.sync_copy(idx_hbm.at[pl.ds(base0, _SC_PER_W)], idx_v)

        gd = [None] * nch
        sd = [None] * nch
        for i in range(nch):
            b = i & 1
            if i >= 2:
                sd[i - 2].wait()
            gd[i] = pltpu.async_copy(
                tab_hbm.at[idx_v.at[pl.ds(i * _SC_CHUNK, _SC_CHUNK)]],
                rows[b], gsem[b])
            if i >= 1:
                gd[i - 1].wait()
                sd[i - 1] = pltpu.async_copy(
                    rows[1 - b],
                    out_hbm.at[pl.ds(base0 + (i - 1) * _SC_CHUNK, _SC_CHUNK)],
                    ssem[1 - b])
        gd[nch - 1].wait()
        last = nch - 1
        sd[last] = pltpu.async_copy(
            rows[last & 1],
            out_hbm.at[pl.ds(base0 + last * _SC_CHUNK, _SC_CHUNK)],
            ssem[last & 1])
        sd[nch - 2].wait()
        sd[last].wait()

    return _sc_gather


NT = 128


def _k3_body(xwp_ref, xwe_ref, xwd_ref, whp_ref, whe_ref, whd_ref,
             wo_ref, bo_ref, pp_ref, pe_ref, pd_ref, h_s, c_s, ys_ref):
    @pl.when(pl.program_id(0) == 0)
    def _init():
        h_s[...] = jnp.zeros_like(h_s)
        c_s[...] = jnp.zeros_like(c_s)

    whp, whe, whd = whp_ref[...], whe_ref[...], whd_ref[...]

    def step(t, carry):
        h, c = carry
        g = jnp.concatenate([
            xwp_ref[:, t, :] + _dot(h[0:16, :], whp),
            xwe_ref[:, t, :] + _dot(h[16:32, :], whe),
            xwd_ref[:, t, :] + _dot(h[32:48, :], whd),
        ], axis=0)
        ig = jax.nn.sigmoid(g[:, 0:128])
        fg = jax.nn.sigmoid(g[:, 128:256])
        gg = jnp.tanh(g[:, 256:384])
        og = jax.nn.sigmoid(g[:, 384:512])
        c2 = fg * c + ig * gg
        h2 = og * jnp.tanh(c2)
        ys_ref[t] = h2
        return (h2, c2)

    h, c = lax.fori_loop(0, NT, step, (h_s[...], c_s[...]), unroll=16)
    h_s[...] = h
    c_s[...] = c

    ys = ys_ref[...]
    pred = jnp.sum(ys * wo_ref[...][None, :, :], axis=2) + bo_ref[...]
    pp_ref[...] = pred[:, 0:16]
    pe_ref[...] = pred[:, 16:32]
    pd_ref[...] = pred[:, 32:48]


def _run_k3(xwp, xwe, xwd, whp, whe, whd, wo_stack, bo_stack):
    xw_spec = pl.BlockSpec((B, NT, G4), lambda i: (0, i, 0))
    w_spec = pl.BlockSpec((M, G4), lambda i: (0, 0))
    p_spec = pl.BlockSpec((NT, B), lambda i: (i, 0))
    return pl.pallas_call(
        _k3_body,
        grid=(T // NT,),
        in_specs=[xw_spec, xw_spec, xw_spec, w_spec, w_spec, w_spec,
                  pl.BlockSpec((3 * B, M), lambda i: (0, 0)),
                  pl.BlockSpec((1, 3 * B), lambda i: (0, 0))],
        out_specs=[p_spec, p_spec, p_spec],
        out_shape=[jax.ShapeDtypeStruct((T, B), f32)] * 3,
        scratch_shapes=[pltpu.VMEM((3 * B, M), f32),
                        pltpu.VMEM((3 * B, M), f32),
                        pltpu.VMEM((NT, 3 * B, M), f32)],
        compiler_params=pltpu.CompilerParams(
            dimension_semantics=("arbitrary",)),
    )(xwp, xwe, xwd, whp, whe, whd, wo_stack, bo_stack)


NCH4 = 8
CH4 = LP // NCH4


def _k4_body(g_ref, sidx_ref, tot_ref, lrt_ref):
    g = g_ref[...]
    c = pl.program_id(0)
    pos3 = (lax.broadcasted_iota(jnp.int32, (CH4, B, 1), 0)
            + c * CH4).astype(f32)
    mask3 = (pos3 < tot_ref[...]).astype(f32)
    mask = mask3.reshape(CH4 * B, 1)
    k = lax.broadcasted_iota(jnp.int32, (1, D_TEXT // 2), 1).astype(f32)
    freqs = jnp.exp(k * NEG_LOG1E4 / float(D_TEXT // 2))
    l8 = lax.broadcasted_iota(jnp.int32, (8, 1), 0).astype(f32)
    ang8 = l8 * freqs
    stab = jnp.concatenate([jnp.sin(ang8), jnp.cos(ang8)], axis=1)
    onehot = (sidx_ref[...] ==
              lax.broadcasted_iota(jnp.int32, (1, 1, 8), 2)).astype(f32)
    sino = _dot(onehot.reshape(CH4 * B, 8), stab)
    lrt_ref[...] = ((g + sino) * mask).reshape(CH4, B, D_TEXT)


def _run_k4(g, sidx_pm, tot):
    return pl.pallas_call(
        _k4_body,
        grid=(NCH4,),
        in_specs=[pl.BlockSpec((CH4 * B, DTAB), lambda c: (c, 0)),
                  pl.BlockSpec((CH4, B, 1), lambda c: (c, 0, 0)),
                  pl.BlockSpec((1, B, 1), lambda c: (0, 0, 0))],
        out_specs=[pl.BlockSpec((CH4, B, D_TEXT), lambda c: (c, 0, 0))],
        out_shape=[jax.ShapeDtypeStruct((L_OUT, B, D_TEXT), f32)],
        compiler_params=pltpu.CompilerParams(
            dimension_semantics=("arbitrary",)),
    )(g, sidx_pm, tot)


def _pred_flat(pp):
    out = [pp['inp']['w'], pp['inp']['b'].reshape(1, M)]
    for lp in pp['layers']:
        out += [lp['mem'].reshape(FILT, M),
                lp['ffn1']['w'], lp['ffn1']['b'].reshape(1, F),
                lp['ffn2']['w'], lp['ffn2']['b'].reshape(1, M)]
    out += [pp['lstm']['W_ih'], pp['lstm']['b'].reshape(1, G4)]
    return out


def kernel(inputs_text_embedding, inputs_emo_embedding, inputs_spk_embedding,
           duration_targets, pitch_targets, energy_targets, params):
    text = inputs_text_embedding
    emo = jnp.swapaxes(inputs_emo_embedding, 1, 2)
    spk = jnp.swapaxes(inputs_spk_embedding, 1, 2)

    dur_tb = duration_targets.astype(jnp.int32)
    pit_tb = pitch_targets
    ene_tb = energy_targets

    weights = ([params['pitch_emb']['w'].reshape(9, D_TEXT),
                params['pitch_emb']['b'].reshape(1, D_TEXT),
                params['energy_emb']['w'].reshape(9, D_TEXT),
                params['energy_emb']['b'].reshape(1, D_TEXT)]
               + _pred_flat(params['pitch_pred'])
               + _pred_flat(params['energy_pred'])
               + [params['dur']['pre1']['w'],
                  params['dur']['pre1']['b'].reshape(1, M),
                  params['dur']['pre2']['w'],
                  params['dur']['pre2']['b'].reshape(1, M),
                  params['dur']['lstm']['W_ih'],
                  params['dur']['lstm']['b'].reshape(1, G4)])

    (xwp, xwe, xwd, table, srcflat, tot, lenr,
     lre_dp, lrs_dp, sidx) = _run_k1(
        text, emo, spk, dur_tb, pit_tb, ene_tb, weights)

    idx_pm = srcflat.reshape(B, LP).T.reshape(NROWS)
    g = _sc_gather_fn()(table, idx_pm)

    def _wo_row(pp):
        return jnp.broadcast_to(pp['out']['w'][:, 0][None, :], (B, M))

    wo_stack = jnp.concatenate([_wo_row(params['pitch_pred']),
                                _wo_row(params['energy_pred']),
                                _wo_row(params['dur'])], axis=0)
    bo_stack = jnp.concatenate(
        [jnp.broadcast_to(params['pitch_pred']['out']['b'].reshape(1, 1), (1, B)),
         jnp.broadcast_to(params['energy_pred']['out']['b'].reshape(1, 1), (1, B)),
         jnp.broadcast_to(params['dur']['out']['b'].reshape(1, 1), (1, B))],
        axis=1)

    ppt, pet, pdt = _run_k3(xwp, xwe, xwd,
                            params['pitch_pred']['lstm']['W_hh'],
                            params['energy_pred']['lstm']['W_hh'],
                            params['dur']['lstm']['W_hh'],
                            wo_stack, bo_stack)

    sidx_pm = sidx.reshape(B, LP).T.reshape(LP, B, 1)
    lrt = _run_k4(g, sidx_pm, tot.reshape(1, B, 1))[0]

    return (jnp.swapaxes(lrt, 0, 1), jnp.swapaxes(lre_dp, 1, 2),
            jnp.swapaxes(lrs_dp, 1, 2), lenr.reshape(B), pdt.T, ppt.T, pet.T)
